# Initial kernel scaffold; baseline (speedup 1.0000x reference)
import jax
import jax.numpy as jnp
from jax.experimental import pallas as pl


def kernel(x, Wg, W1, b1, W2, b2):
    raise NotImplementedError("write your pallas kernel here")


# speedup vs baseline: 1.0964x; 1.0964x over previous
import functools

import jax
import jax.numpy as jnp
from jax.experimental import pallas as pl
from jax.experimental.pallas import tpu as pltpu

_TOP_K = 2
_LB_WEIGHT = 0.01


def _dense_moe_body(x_ref, wgt_ref, w1_ref, b1_ref, w2_ref, b2_ref,
                    out_ref, psum_ref, freq_ref, w_scr, *, num_experts):
    tb = pl.program_id(0)
    e = pl.program_id(1)
    E = num_experts

    @pl.when(e == 0)
    def _router():
        xb = x_ref[...]
        logits = jax.lax.dot_general(
            xb, wgt_ref[...], (((1,), (0,)), ((), ())),
            precision=jax.lax.Precision.DEFAULT,
            preferred_element_type=jnp.float32)
        iota = jax.lax.broadcasted_iota(jnp.int32, logits.shape, 1)
        m1 = jnp.max(logits, axis=-1, keepdims=True)
        i1 = jnp.min(jnp.where(logits == m1, iota, E), axis=-1, keepdims=True)
        masked = jnp.where(iota == i1, -jnp.inf, logits)
        m2 = jnp.max(masked, axis=-1, keepdims=True)
        i2 = jnp.min(jnp.where(masked == m2, iota, E), axis=-1, keepdims=True)
        eb = jnp.exp(m2 - m1)
        w1v = 1.0 / (1.0 + eb)
        w2v = eb / (1.0 + eb)
        w_full = (jnp.where(iota == i1, w1v, 0.0)
                  + jnp.where(iota == i2, w2v, 0.0))
        w_scr[...] = w_full
        p = jnp.exp(logits - m1)
        probs = p / jnp.sum(p, axis=-1, keepdims=True)
        psum_blk = jnp.sum(probs, axis=0, keepdims=True)
        freq_blk = jnp.sum((iota == i1).astype(jnp.float32), axis=0,
                           keepdims=True)

        @pl.when(tb == 0)
        def _():
            psum_ref[...] = psum_blk
            freq_ref[...] = freq_blk

        @pl.when(tb > 0)
        def _():
            psum_ref[...] += psum_blk
            freq_ref[...] += freq_blk

    xb16 = x_ref[...].astype(jnp.bfloat16)
    h = jax.lax.dot_general(
        xb16, w1_ref[0], (((1,), (1,)), ((), ())),
        preferred_element_type=jnp.float32) + b1_ref[0]
    h = jnp.maximum(h, 0.0).astype(jnp.bfloat16)
    y = jax.lax.dot_general(
        h, w2_ref[0], (((1,), (1,)), ((), ())),
        preferred_element_type=jnp.float32) + b2_ref[0]
    iota = jax.lax.broadcasted_iota(jnp.int32, w_scr.shape, 1)
    wcol = jnp.sum(jnp.where(iota == e, w_scr[...], 0.0), axis=-1,
                   keepdims=True)
    contrib = wcol * y

    @pl.when(e == 0)
    def _():
        out_ref[...] = contrib

    @pl.when(e > 0)
    def _():
        out_ref[...] += contrib


def kernel(x, Wg, W1, b1, W2, b2):
    B, S, D = x.shape
    E, F, _ = W1.shape
    N = B * S
    TB = min(512, N)
    n_tb = N // TB

    x2 = x.reshape(N, D)
    wgt = Wg.T.astype(jnp.float32)
    w1c = W1.astype(jnp.bfloat16)
    w2c = W2.astype(jnp.bfloat16)

    grid = (n_tb, E)
    out, psum, freq = pl.pallas_call(
        functools.partial(_dense_moe_body, num_experts=E),
        grid=grid,
        in_specs=[
            pl.BlockSpec((TB, D), lambda tb, e: (tb, 0)),
            pl.BlockSpec((D, E), lambda tb, e: (0, 0)),
            pl.BlockSpec((1, F, D), lambda tb, e: (e, 0, 0)),
            pl.BlockSpec((1, 1, F), lambda tb, e: (e, 0, 0)),
            pl.BlockSpec((1, D, F), lambda tb, e: (e, 0, 0)),
            pl.BlockSpec((1, 1, D), lambda tb, e: (e, 0, 0)),
        ],
        out_specs=[
            pl.BlockSpec((TB, D), lambda tb, e: (tb, 0)),
            pl.BlockSpec((1, E), lambda tb, e: (0, 0)),
            pl.BlockSpec((1, E), lambda tb, e: (0, 0)),
        ],
        out_shape=[
            jax.ShapeDtypeStruct((N, D), jnp.float32),
            jax.ShapeDtypeStruct((1, E), jnp.float32),
            jax.ShapeDtypeStruct((1, E), jnp.float32),
        ],
        scratch_shapes=[pltpu.VMEM((TB, E), jnp.float32)],
        compiler_params=pltpu.CompilerParams(
            dimension_semantics=("arbitrary", "arbitrary")),
    )(x2, wgt, w1c, b1.reshape(E, 1, F), w2c, b2.reshape(E, 1, D))

    avg_probs = psum[0] / N
    fnorm = freq[0] / N
    lb_loss = E * jnp.sum(fnorm * avg_probs)
    return out.reshape(B, S, D), _LB_WEIGHT * lb_loss
